# baseline (speedup 1.0000x reference)
import jax
import jax.numpy as jnp
from jax.experimental import pallas as pl
from jax.experimental.pallas import tpu as pltpu


def _proj_kernel(b_ref, w_ref, bias_ref, h_ref):
    h_ref[...] = (
        jnp.dot(b_ref[...], w_ref[...], preferred_element_type=jnp.float32)
        + bias_ref[...]
    )


def _spmm_kernel(a_ref, h_ref, out_ref):
    out_ref[...] = jnp.dot(
        a_ref[...], h_ref[...], preferred_element_type=jnp.float32
    )


def kernel(a, b, W, bias):
    n, k = a.shape
    d_in = b.shape[1]
    d_out = W.shape[1]
    bias2d = bias.reshape(1, d_out)

    h = pl.pallas_call(
        _proj_kernel,
        out_shape=jax.ShapeDtypeStruct((k, d_out), jnp.float32),
    )(b, W, bias2d)

    bm = 1024
    out = pl.pallas_call(
        _spmm_kernel,
        grid=(n // bm,),
        in_specs=[
            pl.BlockSpec((bm, k), lambda i: (i, 0)),
            pl.BlockSpec((k, d_out), lambda i: (0, 0)),
        ],
        out_specs=pl.BlockSpec((bm, d_out), lambda i: (i, 0)),
        out_shape=jax.ShapeDtypeStruct((n, d_out), jnp.float32),
        compiler_params=pltpu.CompilerParams(
            dimension_semantics=("parallel",),
        ),
    )(a, h)
    return out


# speedup vs baseline: 1.1669x; 1.1669x over previous
import jax
import jax.numpy as jnp
from jax.experimental import pallas as pl
from jax.experimental.pallas import tpu as pltpu


def _proj_kernel(b_ref, w_ref, bias_ref, h_ref):
    h_ref[...] = (
        jnp.dot(b_ref[...], w_ref[...], preferred_element_type=jnp.float32)
        + bias_ref[...]
    )


def _spmm_kernel(a_ref, h_ref, out_ref):
    out_ref[...] = jnp.dot(
        a_ref[...], h_ref[...], preferred_element_type=jnp.float32
    )


def kernel(a, b, W, bias):
    n, k = a.shape
    d_in = b.shape[1]
    d_out = W.shape[1]
    bias2d = bias.reshape(1, d_out)

    h = jnp.dot(b, W) + bias2d

    bm = 512
    out = pl.pallas_call(
        _spmm_kernel,
        grid=(n // bm,),
        in_specs=[
            pl.BlockSpec((bm, k), lambda i: (i, 0)),
            pl.BlockSpec((k, d_out), lambda i: (0, 0)),
        ],
        out_specs=pl.BlockSpec((bm, d_out), lambda i: (i, 0)),
        out_shape=jax.ShapeDtypeStruct((n, d_out), jnp.float32),
        compiler_params=pltpu.CompilerParams(
            dimension_semantics=("parallel",),
        ),
    )(a, h)
    return out


# speedup vs baseline: 1.1758x; 1.0076x over previous
import jax
import jax.numpy as jnp
from jax.experimental import pallas as pl
from jax.experimental.pallas import tpu as pltpu


def _fused_kernel(b_ref, w_ref, bias_ref, a_ref, out_ref, h_ref):
    @pl.when(pl.program_id(0) == 0)
    def _():
        h_ref[...] = (
            jnp.dot(b_ref[...], w_ref[...], preferred_element_type=jnp.float32)
            + bias_ref[...]
        )

    out_ref[...] = jnp.dot(
        a_ref[...], h_ref[...], preferred_element_type=jnp.float32
    )


def kernel(a, b, W, bias):
    n, k = a.shape
    d_in = b.shape[1]
    d_out = W.shape[1]
    bias2d = bias.reshape(1, d_out)

    bm = 512
    out = pl.pallas_call(
        _fused_kernel,
        grid=(n // bm,),
        in_specs=[
            pl.BlockSpec((k, d_in), lambda i: (0, 0)),
            pl.BlockSpec((d_in, d_out), lambda i: (0, 0)),
            pl.BlockSpec((1, d_out), lambda i: (0, 0)),
            pl.BlockSpec((bm, k), lambda i: (i, 0)),
        ],
        out_specs=pl.BlockSpec((bm, d_out), lambda i: (i, 0)),
        out_shape=jax.ShapeDtypeStruct((n, d_out), jnp.float32),
        scratch_shapes=[pltpu.VMEM((k, d_out), jnp.float32)],
        compiler_params=pltpu.CompilerParams(
            dimension_semantics=("arbitrary",),
        ),
    )(b, W, bias2d, a)
    return out
